# baseline (speedup 1.0000x reference)
import functools

import numpy as np

import jax
import jax.numpy as jnp
from jax import lax
from jax.experimental import pallas as pl
from jax.experimental.pallas import tpu as pltpu
from jax.experimental.pallas import tpu_sc as plsc

N = 1024
M = 100000
EPS = 0.05

NW = 32
NG = 8
NS = 4
CSC = 56000
SW = CSC // NS
CC = 280
NCHS = SW // CC
NBUF = 3
TB = 1000
BIG = np.int32(2**30)


def _rotl(x, r):
    return (x << np.uint32(r)) | (x >> np.uint32(32 - r))


def _threefry2x32(k0, k1, x0, x1):
    ks = [np.uint32(k0), np.uint32(k1),
          np.uint32(np.uint32(k0) ^ np.uint32(k1) ^ np.uint32(0x1BD11BDA))]
    rotations = [(13, 15, 26, 6), (17, 29, 16, 24)]
    x0 = x0 + ks[0]
    x1 = x1 + ks[1]
    for r in range(5):
        for rot in rotations[r % 2]:
            x0 = x0 + x1
            x1 = _rotl(x1, rot)
            x1 = x1 ^ x0
        x0 = x0 + ks[(r + 1) % 3]
        x1 = x1 + ks[(r + 2) % 3] + np.uint32(r + 1)
    return x0, x1


def _random_bits(k0, k1, n, chunk=1 << 24):
    old = np.seterr(over="ignore")
    out = np.empty(n, dtype=np.uint32)
    for s in range(0, n, chunk):
        e = min(n, s + chunk)
        lo = np.arange(s, e, dtype=np.uint32)
        hi = np.zeros(e - s, dtype=np.uint32)
        o0, o1 = _threefry2x32(k0, k1, hi, lo)
        out[s:e] = o0 ^ o1
    np.seterr(**old)
    return out


@functools.lru_cache(maxsize=1)
def _sampling_consts():
    bits = _random_bits(0, 1, N * M)
    sampled = np.argmax((bits >> np.uint32(9)).reshape(N, M), axis=1).astype(np.int32)
    bbits = _random_bits(0, 2, N)
    u = ((bbits >> np.uint32(9)).astype(np.float32) * np.float32(2.0**-23))
    b = (u < np.float32(1.0 - EPS)).astype(np.int32)
    return sampled, b


_GDN = lax.GatherDimensionNumbers(
    offset_dims=(), collapsed_slice_dims=(0,), start_index_map=(0,))


def _shuffle(v, perm):
    return lax.gather(v, perm.reshape(16, 1), _GDN, slice_sizes=(1,),
                      mode=lax.GatherScatterMode.PROMISE_IN_BOUNDS)


def _butterfly(v, op, iota16):
    for s in (8, 4, 2, 1):
        v = op(v, _shuffle(v, iota16 ^ s))
    return v


def _sc_body(xt_hbm, scmax_hbm, scidx_hbm,
             buf0, buf1, buf2,
             stg_max, stg_idx, mrg_max, mrg_idx, res_max, res_idx,
             shared_max, shared_idx,
             s0, s1, s2):
    bufs = (buf0, buf1, buf2)
    sems = (s0, s1, s2)
    cid = lax.axis_index("c")
    sid = lax.axis_index("s")
    rgl = sid // NS
    cs = sid % NS
    rg = cid * (NG // 2) + rgl
    row_base = rg * 128 + cs * 32

    col0 = cs * SW

    def src(c):
        return xt_hbm.at[pl.ds(col0 + c * CC, CC), pl.ds(rg * 128, 128)]

    def start(c, b):
        pltpu.make_async_copy(src(c), bufs[b], sems[b]).start()

    def wait(c, b):
        pltpu.make_async_copy(src(c), bufs[b], sems[b]).wait()

    for b in range(NBUF):
        start(b, b)

    vms = [jnp.full((16,), -jnp.inf, jnp.float32) for _ in range(8)]
    vis = [jnp.zeros((16,), jnp.int32) for _ in range(8)]
    ivec = jnp.full((16,), col0, jnp.int32)
    state = (*vms, *vis, ivec)

    def process(b, state):
        def ibody(i, c, b=b):
            vm = list(c[:8])
            vi = list(c[8:16])
            iv = c[16]
            for u in range(2):
                e = i * 2 + u
                for g in range(8):
                    v = bufs[b][e, pl.ds(g * 16, 16)]
                    m = v > vm[g]
                    vm[g] = jnp.where(m, v, vm[g])
                    vi[g] = jnp.where(m, iv, vi[g])
                iv = iv + 1
            return (*vm, *vi, iv)

        return lax.fori_loop(0, CC // 2, ibody, state)

    def kbody(k, state):
        for b in range(NBUF):
            c = k * NBUF + b
            wait(c, b)
            state = process(b, state)

            @pl.when(c + NBUF < NCHS)
            def _(c=c, b=b):
                start(c + NBUF, b)
        return state

    state = lax.fori_loop(0, NCHS // NBUF, kbody, state)
    for c in range(NCHS - NCHS % NBUF, NCHS):
        wait(c, c % NBUF)
        state = process(c % NBUF, state)

    for g in range(8):
        stg_max[pl.ds(g * 16, 16)] = state[g]
        stg_idx[pl.ds(g * 16, 16)] = state[8 + g]
    pltpu.sync_copy(stg_max, shared_max.at[sid])
    pltpu.sync_copy(stg_idx, shared_idx.at[sid])
    plsc.subcore_barrier()
    for k in range(NS):
        pltpu.sync_copy(shared_max.at[rgl * NS + k, pl.ds(cs * 32, 32)],
                        mrg_max.at[k])
        pltpu.sync_copy(shared_idx.at[rgl * NS + k, pl.ds(cs * 32, 32)],
                        mrg_idx.at[k])

    for h in range(2):
        va = mrg_max[0, pl.ds(h * 16, 16)]
        ia = mrg_idx[0, pl.ds(h * 16, 16)]
        for k in range(1, NS):
            vb = mrg_max[k, pl.ds(h * 16, 16)]
            ib = mrg_idx[k, pl.ds(h * 16, 16)]
            m = vb > va
            va = jnp.where(m, vb, va)
            ia = jnp.where(m, ib, ia)
        res_max[pl.ds(h * 16, 16)] = va
        res_idx[pl.ds(h * 16, 16)] = ia

    pltpu.sync_copy(res_max, scmax_hbm.at[pl.ds(row_base, 32)])
    pltpu.sync_copy(res_idx, scidx_hbm.at[pl.ds(row_base, 32)])


@functools.lru_cache(maxsize=1)
def _sc_call():
    mesh = plsc.VectorSubcoreMesh(core_axis_name="c", subcore_axis_name="s")
    return pl.kernel(
        _sc_body,
        mesh=mesh,
        out_type=(jax.ShapeDtypeStruct((N,), jnp.float32),
                  jax.ShapeDtypeStruct((N,), jnp.int32)),
        scratch_types=[
            pltpu.VMEM((CC, 128), jnp.float32),
            pltpu.VMEM((CC, 128), jnp.float32),
            pltpu.VMEM((CC, 128), jnp.float32),
            pltpu.VMEM((128,), jnp.float32),
            pltpu.VMEM((128,), jnp.int32),
            pltpu.VMEM((NS, 32), jnp.float32),
            pltpu.VMEM((NS, 32), jnp.int32),
            pltpu.VMEM((32,), jnp.float32),
            pltpu.VMEM((32,), jnp.int32),
            pltpu.VMEM_SHARED((16, 128), jnp.float32),
            pltpu.VMEM_SHARED((16, 128), jnp.int32),
            pltpu.SemaphoreType.DMA,
            pltpu.SemaphoreType.DMA,
            pltpu.SemaphoreType.DMA,
        ],
    )


def _tc_stripe_body(xt_ref, mx_ref, ix_ref, run_max, run_idx):
    k = pl.program_id(0)
    nb = pl.num_programs(0)

    @pl.when(k == 0)
    def _():
        run_max[...] = jnp.full((8, N), -jnp.inf, jnp.float32)
        run_idx[...] = jnp.zeros((8, N), jnp.int32)

    blk = xt_ref[...]
    rm = run_max[...]
    ri = run_idx[...]
    base = k * (TB // 8)
    for sb in range(TB // 8):
        sub = blk[sb * 8:(sb + 1) * 8, :]
        m = sub > rm
        rm = jnp.where(m, sub, rm)
        ri = jnp.where(m, jnp.full((8, N), base + sb, jnp.int32), ri)
    run_max[...] = rm
    run_idx[...] = ri

    @pl.when(k == nb - 1)
    def _():
        fmax = jnp.max(rm, axis=0, keepdims=True)
        col8 = ri * 8 + lax.broadcasted_iota(jnp.int32, (8, N), 0)
        cand = jnp.where(rm == fmax, col8, jnp.full((8, N), BIG))
        mx_ref[...] = fmax
        ix_ref[...] = jnp.min(cand, axis=0, keepdims=True) + CSC


@functools.lru_cache(maxsize=1)
def _tc_stripe_call():
    return pl.pallas_call(
        _tc_stripe_body,
        grid=((M - CSC) // TB,),
        in_specs=[pl.BlockSpec((TB, N), lambda k: (CSC // TB + k, 0))],
        out_specs=[pl.BlockSpec((1, N), lambda k: (0, 0)),
                   pl.BlockSpec((1, N), lambda k: (0, 0))],
        out_shape=(jax.ShapeDtypeStruct((1, N), jnp.float32),
                   jax.ShapeDtypeStruct((1, N), jnp.int32)),
        scratch_shapes=[pltpu.VMEM((8, N), jnp.float32),
                        pltpu.VMEM((8, N), jnp.int32)],
    )


def _combine_body(scm_ref, sci_ref, tcm_ref, tci_ref, samp_ref, b_ref,
                  out_ref):
    take_tc = tcm_ref[...] > scm_ref[...]
    best = jnp.where(take_tc, tci_ref[...], sci_ref[...])
    sel = b_ref[...] != 0
    out_ref[...] = jnp.where(sel, best, samp_ref[...])


@functools.lru_cache(maxsize=1)
def _combine_call():
    return pl.pallas_call(
        _combine_body,
        out_shape=jax.ShapeDtypeStruct((N, N), jnp.int32),
    )


def kernel(x):
    sampled, b = _sampling_consts()
    xt = x.T
    scm, sci = _sc_call()(xt)
    tcm, tci = _tc_stripe_call()(xt)
    return _combine_call()(
        scm.reshape(N, 1), sci.reshape(N, 1),
        tcm.reshape(N, 1), tci.reshape(N, 1),
        jnp.asarray(sampled).reshape(1, N), jnp.asarray(b).reshape(N, 1))


# speedup vs baseline: 1.0324x; 1.0324x over previous
import functools

import numpy as np

import jax
import jax.numpy as jnp
from jax import lax
from jax.experimental import pallas as pl
from jax.experimental.pallas import tpu as pltpu
from jax.experimental.pallas import tpu_sc as plsc

N = 1024
M = 100000
EPS = 0.05

NW = 32
RG = 256
GN = RG // 16
NG = N // RG
NS = 8
CSC = 56000
SW = CSC // NS
CC = 56
NCHS = SW // CC
NBUF = 3
TB = 1000
BIG = np.int32(2**30)


def _rotl(x, r):
    return (x << np.uint32(r)) | (x >> np.uint32(32 - r))


def _threefry2x32(k0, k1, x0, x1):
    ks = [np.uint32(k0), np.uint32(k1),
          np.uint32(np.uint32(k0) ^ np.uint32(k1) ^ np.uint32(0x1BD11BDA))]
    rotations = [(13, 15, 26, 6), (17, 29, 16, 24)]
    x0 = x0 + ks[0]
    x1 = x1 + ks[1]
    for r in range(5):
        for rot in rotations[r % 2]:
            x0 = x0 + x1
            x1 = _rotl(x1, rot)
            x1 = x1 ^ x0
        x0 = x0 + ks[(r + 1) % 3]
        x1 = x1 + ks[(r + 2) % 3] + np.uint32(r + 1)
    return x0, x1


def _random_bits(k0, k1, n, chunk=1 << 24):
    old = np.seterr(over="ignore")
    out = np.empty(n, dtype=np.uint32)
    for s in range(0, n, chunk):
        e = min(n, s + chunk)
        lo = np.arange(s, e, dtype=np.uint32)
        hi = np.zeros(e - s, dtype=np.uint32)
        o0, o1 = _threefry2x32(k0, k1, hi, lo)
        out[s:e] = o0 ^ o1
    np.seterr(**old)
    return out


@functools.lru_cache(maxsize=1)
def _sampling_consts():
    bits = _random_bits(0, 1, N * M)
    sampled = np.argmax((bits >> np.uint32(9)).reshape(N, M), axis=1).astype(np.int32)
    bbits = _random_bits(0, 2, N)
    u = ((bbits >> np.uint32(9)).astype(np.float32) * np.float32(2.0**-23))
    b = (u < np.float32(1.0 - EPS)).astype(np.int32)
    return sampled, b


_GDN = lax.GatherDimensionNumbers(
    offset_dims=(), collapsed_slice_dims=(0,), start_index_map=(0,))


def _shuffle(v, perm):
    return lax.gather(v, perm.reshape(16, 1), _GDN, slice_sizes=(1,),
                      mode=lax.GatherScatterMode.PROMISE_IN_BOUNDS)


def _butterfly(v, op, iota16):
    for s in (8, 4, 2, 1):
        v = op(v, _shuffle(v, iota16 ^ s))
    return v


def _sc_body(xt_hbm, scmax_hbm, scidx_hbm,
             buf0, buf1, buf2,
             stg_max, stg_idx, mrg_max, mrg_idx, res_max, res_idx,
             shared_max, shared_idx,
             s0, s1, s2):
    bufs = (buf0, buf1, buf2)
    sems = (s0, s1, s2)
    cid = lax.axis_index("c")
    sid = lax.axis_index("s")
    rgl = sid // NS
    cs = sid % NS
    rg = cid * (NG // 2) + rgl
    row_base = rg * RG + cs * 32

    col0 = cs * SW

    def src(c):
        return xt_hbm.at[pl.ds(col0 + c * CC, CC), pl.ds(rg * RG, RG)]

    def start(c, b):
        pltpu.make_async_copy(src(c), bufs[b], sems[b]).start()

    def wait(c, b):
        pltpu.make_async_copy(src(c), bufs[b], sems[b]).wait()

    for b in range(NBUF):
        start(b, b)

    vms = [jnp.full((16,), -jnp.inf, jnp.float32) for _ in range(GN)]
    vis = [jnp.zeros((16,), jnp.int32) for _ in range(GN)]
    ivec = jnp.full((16,), col0, jnp.int32)
    state = (*vms, *vis, ivec)

    def process(b, state):
        def ibody(e, c, b=b):
            vm = list(c[:GN])
            vi = list(c[GN:2 * GN])
            iv = c[2 * GN]
            for g in range(GN):
                v = bufs[b][e, pl.ds(g * 16, 16)]
                m = v > vm[g]
                vm[g] = jnp.where(m, v, vm[g])
                vi[g] = jnp.where(m, iv, vi[g])
            return (*vm, *vi, iv + 1)

        return lax.fori_loop(0, CC, ibody, state)

    def kbody(k, state):
        for b in range(NBUF):
            c = k * NBUF + b
            wait(c, b)
            state = process(b, state)

            @pl.when(c + NBUF < NCHS)
            def _(c=c, b=b):
                start(c + NBUF, b)
        return state

    state = lax.fori_loop(0, NCHS // NBUF, kbody, state)
    for c in range(NCHS - NCHS % NBUF, NCHS):
        wait(c, c % NBUF)
        state = process(c % NBUF, state)

    for g in range(GN):
        stg_max[pl.ds(g * 16, 16)] = state[g]
        stg_idx[pl.ds(g * 16, 16)] = state[GN + g]
    pltpu.sync_copy(stg_max, shared_max.at[sid])
    pltpu.sync_copy(stg_idx, shared_idx.at[sid])
    plsc.subcore_barrier()
    for k in range(NS):
        pltpu.sync_copy(shared_max.at[rgl * NS + k, pl.ds(cs * 32, 32)],
                        mrg_max.at[k])
        pltpu.sync_copy(shared_idx.at[rgl * NS + k, pl.ds(cs * 32, 32)],
                        mrg_idx.at[k])

    for h in range(2):
        va = mrg_max[0, pl.ds(h * 16, 16)]
        ia = mrg_idx[0, pl.ds(h * 16, 16)]
        for k in range(1, NS):
            vb = mrg_max[k, pl.ds(h * 16, 16)]
            ib = mrg_idx[k, pl.ds(h * 16, 16)]
            m = vb > va
            va = jnp.where(m, vb, va)
            ia = jnp.where(m, ib, ia)
        res_max[pl.ds(h * 16, 16)] = va
        res_idx[pl.ds(h * 16, 16)] = ia

    pltpu.sync_copy(res_max, scmax_hbm.at[pl.ds(row_base, 32)])
    pltpu.sync_copy(res_idx, scidx_hbm.at[pl.ds(row_base, 32)])


@functools.lru_cache(maxsize=1)
def _sc_call():
    mesh = plsc.VectorSubcoreMesh(core_axis_name="c", subcore_axis_name="s")
    return pl.kernel(
        _sc_body,
        mesh=mesh,
        out_type=(jax.ShapeDtypeStruct((N,), jnp.float32),
                  jax.ShapeDtypeStruct((N,), jnp.int32)),
        scratch_types=[
            pltpu.VMEM((CC, RG), jnp.float32),
            pltpu.VMEM((CC, RG), jnp.float32),
            pltpu.VMEM((CC, RG), jnp.float32),
            pltpu.VMEM((RG,), jnp.float32),
            pltpu.VMEM((RG,), jnp.int32),
            pltpu.VMEM((NS, 32), jnp.float32),
            pltpu.VMEM((NS, 32), jnp.int32),
            pltpu.VMEM((32,), jnp.float32),
            pltpu.VMEM((32,), jnp.int32),
            pltpu.VMEM_SHARED((16, RG), jnp.float32),
            pltpu.VMEM_SHARED((16, RG), jnp.int32),
            pltpu.SemaphoreType.DMA,
            pltpu.SemaphoreType.DMA,
            pltpu.SemaphoreType.DMA,
        ],
    )


def _tc_stripe_body(xt_ref, mx_ref, ix_ref, run_max, run_idx):
    k = pl.program_id(0)
    nb = pl.num_programs(0)

    @pl.when(k == 0)
    def _():
        run_max[...] = jnp.full((8, N), -jnp.inf, jnp.float32)
        run_idx[...] = jnp.zeros((8, N), jnp.int32)

    blk = xt_ref[...]
    rm = run_max[...]
    ri = run_idx[...]
    base = k * (TB // 8)
    for sb in range(TB // 8):
        sub = blk[sb * 8:(sb + 1) * 8, :]
        m = sub > rm
        rm = jnp.where(m, sub, rm)
        ri = jnp.where(m, jnp.full((8, N), base + sb, jnp.int32), ri)
    run_max[...] = rm
    run_idx[...] = ri

    @pl.when(k == nb - 1)
    def _():
        fmax = jnp.max(rm, axis=0, keepdims=True)
        col8 = ri * 8 + lax.broadcasted_iota(jnp.int32, (8, N), 0)
        cand = jnp.where(rm == fmax, col8, jnp.full((8, N), BIG))
        mx_ref[...] = fmax
        ix_ref[...] = jnp.min(cand, axis=0, keepdims=True) + CSC


@functools.lru_cache(maxsize=1)
def _tc_stripe_call():
    return pl.pallas_call(
        _tc_stripe_body,
        grid=((M - CSC) // TB,),
        in_specs=[pl.BlockSpec((TB, N), lambda k: (CSC // TB + k, 0))],
        out_specs=[pl.BlockSpec((1, N), lambda k: (0, 0)),
                   pl.BlockSpec((1, N), lambda k: (0, 0))],
        out_shape=(jax.ShapeDtypeStruct((1, N), jnp.float32),
                   jax.ShapeDtypeStruct((1, N), jnp.int32)),
        scratch_shapes=[pltpu.VMEM((8, N), jnp.float32),
                        pltpu.VMEM((8, N), jnp.int32)],
    )


def _combine_body(scm_ref, sci_ref, tcm_ref, tci_ref, samp_ref, b_ref,
                  out_ref):
    take_tc = tcm_ref[...] > scm_ref[...]
    best = jnp.where(take_tc, tci_ref[...], sci_ref[...])
    sel = b_ref[...] != 0
    out_ref[...] = jnp.where(sel, best, samp_ref[...])


@functools.lru_cache(maxsize=1)
def _combine_call():
    return pl.pallas_call(
        _combine_body,
        out_shape=jax.ShapeDtypeStruct((N, N), jnp.int32),
    )


def kernel(x):
    sampled, b = _sampling_consts()
    xt = x.T
    scm, sci = _sc_call()(xt)
    tcm, tci = _tc_stripe_call()(xt)
    return _combine_call()(
        scm.reshape(N, 1), sci.reshape(N, 1),
        tcm.reshape(N, 1), tci.reshape(N, 1),
        jnp.asarray(sampled).reshape(1, N), jnp.asarray(b).reshape(N, 1))
